# Initial kernel scaffold; baseline (speedup 1.0000x reference)
import jax
import jax.numpy as jnp
from jax.experimental import pallas as pl


def kernel(type, update_step, requests, edge_index, latency, batch, emb, w0, as0, ad0, b0, w1, as1, ad1, b1, w2, as2, ad2, b2, w3, as3, ad3, b3, cw1, cb1, cw2, cb2, cw3, cb3):
    raise NotImplementedError("write your pallas kernel here")


# speedup vs baseline: 209.2160x; 209.2160x over previous
import functools

import jax
import jax.numpy as jnp
from jax import lax
from jax.experimental import pallas as pl
from jax.experimental.pallas import tpu as pltpu
from jax.experimental.pallas import tpu_sc as plsc

N = 100000
NP = 100096
E = 3200000
ET = E + N
CH = 1024
NW = 32
PERW = 103424
NCH = PERW // CH
ETP = PERW * NW
H = 4
C = 3
HC = 12
FC = 128
NG = 16
BN = 5000
GRID = N // BN
RPS = NP // 16


def _stats_body(req_ref, out_ref):
    r = req_ref[...]
    fi = (lax.broadcasted_iota(jnp.int32, (12500, 8), 0) * 8
          + lax.broadcasted_iota(jnp.int32, (12500, 8), 1))
    msk = (fi >= 15).astype(jnp.float32)
    cnt = jnp.float32(N - 15)
    s1 = jnp.sum(r * msk)
    s2 = jnp.sum(r * r * msk)
    mean = s1 / cnt
    var = (s2 - cnt * mean * mean) / (cnt - 1.0)
    rstd = 1.0 / (jnp.sqrt(jnp.maximum(var, 0.0)) + 1e-6)
    li = (lax.broadcasted_iota(jnp.int32, (8, 128), 0) * 128
          + lax.broadcasted_iota(jnp.int32, (8, 128), 1))
    out_ref[...] = jnp.where(li == 0, mean, jnp.where(li == 1, rstd, 0.0))


def _stats(req2):
    return pl.pallas_call(
        _stats_body,
        out_shape=jax.ShapeDtypeStruct((8, 128), jnp.float32),
    )(req2)


_TBL_SHAPES = [
    jax.ShapeDtypeStruct((N, 8), jnp.float32),
    jax.ShapeDtypeStruct((N, 8), jnp.float32),
    jax.ShapeDtypeStruct((N, 8), jnp.float32),
    jax.ShapeDtypeStruct((GRID, 1, 16), jnp.float32),
]
_TBL_SPECS = [
    pl.BlockSpec((BN, 8), lambda j: (j, 0)),
    pl.BlockSpec((BN, 8), lambda j: (j, 0)),
    pl.BlockSpec((BN, 8), lambda j: (j, 0)),
    pl.BlockSpec((1, 1, 16), lambda j: (j, 0, 0)),
]


def _emit_tables(h, als, ald, ta_ref, tb_ref, bt_ref, mx_ref):
    ta_ref[...] = jnp.concatenate([h[:, 0:6], als[:, 0:2]], axis=1)
    tb_ref[...] = jnp.concatenate([h[:, 6:12], als[:, 2:4]], axis=1)
    bt_ref[...] = jnp.concatenate(
        [ald, jnp.zeros((ald.shape[0], 4), jnp.float32)], axis=1)
    mals = jnp.max(als, axis=0)
    mald = jnp.max(ald, axis=0)
    row = jnp.concatenate([mals, mald, jnp.full((8,), -1e30, jnp.float32)])
    mx_ref[...] = row[None, None, :]


def _prep0_body(type_ref, upd_ref, req_ref, stats_ref, emb_ref, w_ref, ms_ref,
                md_ref, ta_ref, tb_ref, bt_ref, mx_ref):
    j = pl.program_id(0)
    tid = type_ref[...]
    oh = (tid == lax.broadcasted_iota(jnp.int32, (1, 4), 1)).astype(jnp.float32)
    x3 = jnp.dot(oh, emb_ref[...], preferred_element_type=jnp.float32)
    mean = stats_ref[0, 0]
    rstd = stats_ref[0, 1]
    r = req_ref[...]
    gidx = j * BN + lax.broadcasted_iota(jnp.int32, (BN, 1), 0)
    rf = jnp.where(gidx < 15, r, (r - mean) * rstd)
    x = jnp.concatenate([x3, rf, upd_ref[...]], axis=1)
    hmat = jnp.dot(x, w_ref[...], preferred_element_type=jnp.float32)
    als = jnp.dot(hmat, ms_ref[...], preferred_element_type=jnp.float32)
    ald = jnp.dot(hmat, md_ref[...], preferred_element_type=jnp.float32)
    _emit_tables(hmat, als, ald, ta_ref, tb_ref, bt_ref, mx_ref)


def _prep0(typec, upd, req1, stats, emb, w, ms, md):
    return pl.pallas_call(
        _prep0_body,
        grid=(GRID,),
        in_specs=[
            pl.BlockSpec((BN, 1), lambda j: (j, 0)),
            pl.BlockSpec((BN, 1), lambda j: (j, 0)),
            pl.BlockSpec((BN, 1), lambda j: (j, 0)),
            pl.BlockSpec((8, 128), lambda j: (0, 0)),
            pl.BlockSpec((4, 3), lambda j: (0, 0)),
            pl.BlockSpec((5, HC), lambda j: (0, 0)),
            pl.BlockSpec((HC, H), lambda j: (0, 0)),
            pl.BlockSpec((HC, H), lambda j: (0, 0)),
        ],
        out_specs=_TBL_SPECS,
        out_shape=_TBL_SHAPES,
    )(typec, upd, req1, stats, emb, w, ms, md)


def _merge_x(aa, ab, r4, b):
    num = jnp.concatenate([aa[:, 0:6], ab[:, 0:6]], axis=1)
    den = jnp.concatenate([aa[:, 6:8], ab[:, 6:8]], axis=1)
    den3 = jnp.dot(den, r4, preferred_element_type=jnp.float32)
    return num / den3 + b


def _prep_body(acca_ref, accb_ref, b_ref, r4_ref, w_ref, ms_ref, md_ref,
               ta_ref, tb_ref, bt_ref, mx_ref):
    aa = acca_ref[0] + acca_ref[1]
    ab = accb_ref[0] + accb_ref[1]
    x = jnp.maximum(_merge_x(aa, ab, r4_ref[...], b_ref[...]), 0.0)
    hmat = jnp.dot(x, w_ref[...], preferred_element_type=jnp.float32)
    als = jnp.dot(hmat, ms_ref[...], preferred_element_type=jnp.float32)
    ald = jnp.dot(hmat, md_ref[...], preferred_element_type=jnp.float32)
    _emit_tables(hmat, als, ald, ta_ref, tb_ref, bt_ref, mx_ref)


def _prep(acca, accb, b, r4, w, ms, md):
    return pl.pallas_call(
        _prep_body,
        grid=(GRID,),
        in_specs=[
            pl.BlockSpec((2, BN, 8), lambda j: (0, j, 0)),
            pl.BlockSpec((2, BN, 8), lambda j: (0, j, 0)),
            pl.BlockSpec((1, HC), lambda j: (0, 0)),
            pl.BlockSpec((H, HC), lambda j: (0, 0)),
            pl.BlockSpec((HC, HC), lambda j: (0, 0)),
            pl.BlockSpec((HC, H), lambda j: (0, 0)),
            pl.BlockSpec((HC, H), lambda j: (0, 0)),
        ],
        out_specs=_TBL_SPECS,
        out_shape=_TBL_SHAPES,
    )(acca, accb, b, r4, w, ms, md)


def _final_body(acca_ref, accb_ref, b_ref, r4_ref, cw1_ref, cb1_ref, cw2_ref,
                cb2_ref, cw3_ref, cb3_ref, batch_ref, o_ref, res_ref):
    j = pl.program_id(0)
    aa = acca_ref[0] + acca_ref[1]
    ab = accb_ref[0] + accb_ref[1]
    x = _merge_x(aa, ab, r4_ref[...], b_ref[...])
    h1 = jnp.maximum(
        jnp.dot(x, cw1_ref[...], preferred_element_type=jnp.float32)
        + cb1_ref[...], 0.0)
    h2 = jnp.maximum(
        jnp.dot(h1, cw2_ref[...], preferred_element_type=jnp.float32)
        + cb2_ref[...], 0.0)
    nv = jnp.maximum(
        jnp.dot(h2, cw3_ref[...], preferred_element_type=jnp.float32)
        + cb3_ref[0, 0], 0.0)
    oh = (batch_ref[...] == lax.broadcasted_iota(jnp.int32, (1, NG), 1)
          ).astype(jnp.float32)
    sums = lax.dot_general(oh, nv, (((0,), (0,)), ((), ())),
                           preferred_element_type=jnp.float32)
    cnt = jnp.sum(oh, axis=0)[:, None]
    contrib = jnp.concatenate(
        [sums, cnt, jnp.zeros((NG, 126), jnp.float32)], axis=1)
    prev = jnp.where(j == 0, jnp.zeros_like(contrib), o_ref[...])
    tot = prev + contrib
    o_ref[...] = tot

    @pl.when(j == GRID - 1)
    def _():
        res_ref[...] = tot[:, 0:1] / jnp.maximum(tot[:, 1:2], 1.0)


def _final(acca, accb, b, r4, cw1, cb1, cw2, cb2, cw3, cb3p, batch2):
    _, res = pl.pallas_call(
        _final_body,
        grid=(GRID,),
        in_specs=[
            pl.BlockSpec((2, BN, 8), lambda j: (0, j, 0)),
            pl.BlockSpec((2, BN, 8), lambda j: (0, j, 0)),
            pl.BlockSpec((1, HC), lambda j: (0, 0)),
            pl.BlockSpec((H, HC), lambda j: (0, 0)),
            pl.BlockSpec((HC, FC), lambda j: (0, 0)),
            pl.BlockSpec((1, FC), lambda j: (0, 0)),
            pl.BlockSpec((FC, FC), lambda j: (0, 0)),
            pl.BlockSpec((1, FC), lambda j: (0, 0)),
            pl.BlockSpec((FC, 1), lambda j: (0, 0)),
            pl.BlockSpec((8, 128), lambda j: (0, 0)),
            pl.BlockSpec((BN, 1), lambda j: (j, 0)),
        ],
        out_specs=[
            pl.BlockSpec((NG, 128), lambda j: (0, 0)),
            pl.BlockSpec((NG, 1), lambda j: (0, 0)),
        ],
        out_shape=[
            jax.ShapeDtypeStruct((NG, 128), jnp.float32),
            jax.ShapeDtypeStruct((NG, 1), jnp.float32),
        ],
    )(acca, accb, b, r4, cw1, cb1, cw2, cb2, cw3, cb3p, batch2)
    return res


_MESH = plsc.VectorSubcoreMesh(core_axis_name="c", subcore_axis_name="s")


def _sc_body(p, src_ref, dst_ref, t_ref, bt_ref, mx_ref, acc_ref,
             sidx, didx, abuf, bbuf, obuf, mxv, gst, accsh, sem, sem2):
    c = lax.axis_index("c")
    s = lax.axis_index("s")
    wid = c * 16 + s
    iota = lax.broadcasted_iota(jnp.int32, (16,), 0)

    pltpu.sync_copy(mx_ref, mxv)
    m = jnp.full((16,), -1e30, jnp.float32)
    for k in range(GRID):
        m = jnp.maximum(m, mxv[pl.ds(16 * k, 16)])
    gst[pl.ds(16, 16)] = m
    ghs = []
    for hh in range(2):
        head = 2 * p + hh
        sh = plsc.load_gather(gst, [jnp.full((16,), 16 + head, jnp.int32)])
        dh = plsc.load_gather(gst, [jnp.full((16,), 20 + head, jnp.int32)])
        tt = sh + dh
        ghs.append(jnp.where(tt < 0, tt * 0.2, tt))

    zeros16 = jnp.zeros((16,), jnp.float32)
    rdiv = lax.shift_right_logical(iota, 3)
    cmod = lax.bitwise_and(iota, 7)

    def _zrow(r, carry):
        plsc.store_scatter(obuf, [2 * r + rdiv, cmod], zeros16)
        return carry
    lax.fori_loop(0, CH // 2, _zrow, 0)
    base_row = s * RPS
    nfull = RPS // CH
    for k in range(nfull):
        pltpu.sync_copy(obuf, accsh.at[pl.ds(base_row + k * CH, CH)])
    rem = RPS - nfull * CH
    pltpu.sync_copy(obuf.at[pl.ds(0, rem)],
                    accsh.at[pl.ds(base_row + nfull * CH, rem)])
    plsc.subcore_barrier()

    ets = jnp.int32(ET)

    def _chunk(i, carry):
        rowb = wid * (PERW // 128) + i * (CH // 128)
        pltpu.sync_copy(src_ref.at[pl.ds(rowb, CH // 128)], sidx)
        pltpu.sync_copy(dst_ref.at[pl.ds(rowb, CH // 128)], didx)
        cps = []
        for jj in range(CH // 128):
            cps.append(pltpu.async_copy(
                t_ref.at[sidx.at[jj]], abuf.at[pl.ds(jj * 128, 128)], sem))
            cps.append(pltpu.async_copy(
                bt_ref.at[didx.at[jj]], bbuf.at[pl.ds(jj * 128, 128)], sem2))
        for cp in cps:
            cp.wait()
        ebase = wid * PERW + i * CH

        def _grp(g, carry2):
            evec = iota + g * 16
            valid = (evec + ebase) < ets
            for hh in range(2):
                acol = jnp.full((16,), 6 + hh, jnp.int32)
                a_h = plsc.load_gather(abuf, [evec, acol])
                d_h = plsc.load_gather(
                    bbuf, [evec, jnp.full((16,), 2 * p + hh, jnp.int32)])
                tt = a_h + d_h
                tt = jnp.where(tt < 0, tt * 0.2, tt) - ghs[hh]
                exh = jnp.exp(jnp.maximum(tt, -80.0))
                exh = jnp.where(valid, exh, 0.0)
                plsc.store_scatter(obuf, [evec, acol], exh)
                for c3 in range(C):
                    cv = jnp.full((16,), C * hh + c3, jnp.int32)
                    v = plsc.load_gather(abuf, [evec, cv]) * exh
                    plsc.store_scatter(obuf, [evec, cv], v)
            return carry2
        lax.fori_loop(0, CH // 16, _grp, 0)
        for jj in range(CH // 128):
            pltpu.sync_copy(obuf.at[pl.ds(jj * 128, 128)],
                            accsh.at[didx.at[jj]], add=True)
        return carry
    lax.fori_loop(0, NCH, _chunk, 0)

    plsc.subcore_barrier()
    pltpu.sync_copy(accsh.at[pl.ds(base_row, RPS)],
                    acc_ref.at[c, pl.ds(base_row, RPS)])


def _make_sc(p):
    @functools.partial(
        pl.kernel,
        out_type=jax.ShapeDtypeStruct((2, NP, 8), jnp.float32),
        mesh=_MESH,
        compiler_params=pltpu.CompilerParams(use_tc_tiling_on_sc=False,
                                             needs_layout_passes=False),
        scratch_types=[
            pltpu.VMEM((CH // 128, 128), jnp.int32),
            pltpu.VMEM((CH // 128, 128), jnp.int32),
            pltpu.VMEM((CH, 8), jnp.float32),
            pltpu.VMEM((CH, 8), jnp.float32),
            pltpu.VMEM((CH, 8), jnp.float32),
            pltpu.VMEM((GRID * 16,), jnp.float32),
            pltpu.VMEM((32,), jnp.float32),
            pltpu.VMEM_SHARED((NP, 8), jnp.float32),
            pltpu.SemaphoreType.DMA,
            pltpu.SemaphoreType.DMA,
        ],
        name=f"sc_edge_pass{p}",
    )
    def _k(src_ref, dst_ref, t_ref, bt_ref, mx_ref, acc_ref,
           sidx, didx, abuf, bbuf, obuf, mxv, gst, accsh, sem, sem2):
        _sc_body(p, src_ref, dst_ref, t_ref, bt_ref, mx_ref, acc_ref,
                 sidx, didx, abuf, bbuf, obuf, mxv, gst, accsh, sem, sem2)
    return _k


_SC_PASS = (_make_sc(0), _make_sc(1))


def _head_mat(a):
    m = jnp.zeros((HC, H), jnp.float32)
    return m.at[jnp.arange(HC), jnp.repeat(jnp.arange(H), C)].set(
        a.reshape(HC).astype(jnp.float32))


def kernel(type, update_step, requests, edge_index, latency, batch, emb,
           w0, as0, ad0, b0, w1, as1, ad1, b1, w2, as2, ad2, b2,
           w3, as3, ad3, b3, cw1, cb1, cw2, cb2, cw3, cb3):
    f32 = jnp.float32
    typec = type.astype(jnp.int32).reshape(N, 1)
    upd = update_step.astype(f32).reshape(N, 1)
    req1 = requests.astype(f32).reshape(N, 1)
    req2 = requests.astype(f32).reshape(12500, 8)
    batch2 = batch.astype(jnp.int32).reshape(N, 1)

    ar = jnp.arange(N, dtype=jnp.int32)
    pad = jnp.zeros((ETP - ET,), jnp.int32)
    srcp = jnp.concatenate([edge_index[0].astype(jnp.int32), ar, pad]
                           ).reshape(ETP // 128, 128)
    dstp = jnp.concatenate([edge_index[1].astype(jnp.int32), ar, pad]
                           ).reshape(ETP // 128, 128)

    r4 = jnp.zeros((H, HC), f32).at[
        jnp.repeat(jnp.arange(H), C), jnp.arange(HC)].set(1.0)
    cb3p = jnp.zeros((8, 128), f32).at[0, 0].set(cb3.astype(f32)[0])

    stats = _stats(req2)
    ta, tb, bt, mx = _prep0(typec, upd, req1, stats, emb.astype(f32),
                            w0.astype(f32), _head_mat(as0), _head_mat(ad0))
    mxf = mx.reshape(GRID * 16)
    acca = _SC_PASS[0](srcp, dstp, ta, bt, mxf)
    accb = _SC_PASS[1](srcp, dstp, tb, bt, mxf)
    for w, a_s, a_d, b_prev in ((w1, as1, ad1, b0), (w2, as2, ad2, b1),
                                (w3, as3, ad3, b2)):
        ta, tb, bt, mx = _prep(acca, accb, b_prev.astype(f32).reshape(1, HC),
                               r4, w.astype(f32), _head_mat(a_s),
                               _head_mat(a_d))
        mxf = mx.reshape(GRID * 16)
        acca = _SC_PASS[0](srcp, dstp, ta, bt, mxf)
        accb = _SC_PASS[1](srcp, dstp, tb, bt, mxf)
    return _final(acca, accb, b3.astype(f32).reshape(1, HC), r4,
                  cw1.astype(f32), cb1.astype(f32).reshape(1, FC),
                  cw2.astype(f32), cb2.astype(f32).reshape(1, FC),
                  cw3.astype(f32), cb3p, batch2)
